# baseline (speedup 1.0000x reference)
import functools
import jax
import jax.numpy as jnp
from jax import lax
from jax.experimental import pallas as pl
from jax.experimental.pallas import tpu as pltpu
from jax.experimental.pallas import tpu_sc as plsc

_B, _S, _D = 4, 2048, 1024
_NC, _NS = 2, 16
_NW = _NC * _NS
_ROWS_W = _S // _NW
_CS = 8
_CHUNKS = _ROWS_W // _CS
_CHUNK_F = _CS * _D
_NVEC = _CHUNK_F // 16


def _sc_body(x_hbm, pos_hbm, out_hbm, pos_v, x0, x1, x2, x3):
    w = lax.axis_index("s") * _NC + lax.axis_index("c")
    base_row = w * _ROWS_W
    x_bufs = (x0, x1, x2, x3)

    def chunk_body(c, carry):
        pos_off = (base_row + c * _CS) * _D
        pltpu.sync_copy(pos_hbm.at[pl.ds(pos_off, _CHUNK_F)], pos_v)
        for b in range(_B):
            pltpu.sync_copy(
                x_hbm.at[pl.ds(b * _S * _D + pos_off, _CHUNK_F)], x_bufs[b])

        def vec_body(i, c2):
            sl = pl.ds(i * 16, 16)
            p = pos_v[sl]
            for b in range(_B):
                x_bufs[b][sl] = x_bufs[b][sl] + p
            return c2

        lax.fori_loop(0, _NVEC, vec_body, 0)
        for b in range(_B):
            pltpu.sync_copy(
                x_bufs[b], out_hbm.at[pl.ds(b * _S * _D + pos_off, _CHUNK_F)])
        return carry

    lax.fori_loop(0, _CHUNKS, chunk_body, 0)


_sc_add = functools.partial(
    pl.kernel,
    mesh=plsc.VectorSubcoreMesh(core_axis_name="c", subcore_axis_name="s"),
    out_type=jax.ShapeDtypeStruct((_B * _S * _D,), jnp.float32),
    scratch_types=[pltpu.VMEM((_CHUNK_F,), jnp.float32) for _ in range(5)],
)(_sc_body)


def kernel(x, pos_embedding):
    B, S, D = x.shape
    out = _sc_add(x.reshape(-1), pos_embedding[:S].reshape(-1))
    return out.reshape(B, S, D)


# speedup vs baseline: 1.9749x; 1.9749x over previous
import functools
import jax
import jax.numpy as jnp
from jax import lax
from jax.experimental import pallas as pl
from jax.experimental.pallas import tpu as pltpu
from jax.experimental.pallas import tpu_sc as plsc

_B, _S, _D = 4, 2048, 1024
_NC, _NS = 2, 16
_NW = _NC * _NS
_ROWS_W = _S // _NW
_CS = 8
_CHUNKS = _ROWS_W // _CS
_NCOL = _D // 16


def _sc_body(x_hbm, pos_hbm, out_hbm, pos_v, x0, x1, x2, x3):
    w = lax.axis_index("s") * _NC + lax.axis_index("c")
    base_row = w * _ROWS_W
    x_bufs = (x0, x1, x2, x3)

    def chunk_body(c, carry):
        row0 = base_row + c * _CS
        pltpu.sync_copy(pos_hbm.at[pl.ds(row0, _CS), :], pos_v)
        for b in range(_B):
            pltpu.sync_copy(x_hbm.at[pl.ds(b * _S + row0, _CS), :], x_bufs[b])

        for r in range(_CS):
            @plsc.parallel_loop(0, _NCOL, 1, unroll=4)
            def col_body(j):
                sl = pl.ds(j * 16, 16)
                p = pos_v[r, sl]
                for b in range(_B):
                    x_bufs[b][r, sl] = x_bufs[b][r, sl] + p

        for b in range(_B):
            pltpu.sync_copy(x_bufs[b], out_hbm.at[pl.ds(b * _S + row0, _CS), :])
        return carry

    lax.fori_loop(0, _CHUNKS, chunk_body, 0)


_sc_add = functools.partial(
    pl.kernel,
    mesh=plsc.VectorSubcoreMesh(core_axis_name="c", subcore_axis_name="s"),
    out_type=jax.ShapeDtypeStruct((_B * _S, _D), jnp.float32),
    scratch_types=[pltpu.VMEM((_CS, _D), jnp.float32) for _ in range(5)],
    compiler_params=pltpu.CompilerParams(use_tc_tiling_on_sc=True),
)(_sc_body)


def kernel(x, pos_embedding):
    B, S, D = x.shape
    out = _sc_add(x.reshape(B * S, D), pos_embedding[:S])
    return out.reshape(B, S, D)


# speedup vs baseline: 2.9313x; 1.4843x over previous
import functools
import jax
import jax.numpy as jnp
from jax import lax
from jax.experimental import pallas as pl
from jax.experimental.pallas import tpu as pltpu
from jax.experimental.pallas import tpu_sc as plsc

_B, _S, _D = 4, 2048, 1024
_NC, _NS = 2, 16
_NW = _NC * _NS
_ROWS_W = _S // _NW
_CS = 8
_CHUNKS = _ROWS_W // _CS
_NSETS = 3
_NCOL = _D // 16


def _sc_body(x_hbm, pos_hbm, out_hbm, *scratch):
    bufs = scratch[: _NSETS * 5]
    sems = scratch[_NSETS * 5:]
    sets = []
    for s in range(_NSETS):
        pos_v = bufs[s * 5]
        xb = bufs[s * 5 + 1: s * 5 + 5]
        sets.append((pos_v, xb, sems[2 * s], sems[2 * s + 1]))

    w = lax.axis_index("s") * _NC + lax.axis_index("c")
    base_row = w * _ROWS_W

    def issue_in(c, s):
        pos_v, xb, sin, _ = sets[s]
        row0 = base_row + c * _CS
        ds = [pltpu.async_copy(pos_hbm.at[pl.ds(row0, _CS), :], pos_v, sin)]
        for b in range(_B):
            ds.append(pltpu.async_copy(
                x_hbm.at[pl.ds(b * _S + row0, _CS), :], xb[b], sin))
        return ds

    def issue_out(c, s):
        _, xb, _, sout = sets[s]
        row0 = base_row + c * _CS
        return [pltpu.async_copy(
            xb[b], out_hbm.at[pl.ds(b * _S + row0, _CS), :], sout)
            for b in range(_B)]

    def compute(s):
        pos_v, xb, _, _ = sets[s]
        for r in range(_CS):
            @plsc.parallel_loop(0, _NCOL, 1, unroll=4)
            def col_body(j):
                sl = pl.ds(j * 16, 16)
                p = pos_v[r, sl]
                for b in range(_B):
                    xb[b][r, sl] = xb[b][r, sl] + p

    in_d = {}
    out_d = {}
    for c in range(_NSETS):
        in_d[c] = issue_in(c, c % _NSETS)
    for c in range(_CHUNKS):
        s = c % _NSETS
        for d in in_d.pop(c):
            d.wait()
        compute(s)
        out_d[c] = issue_out(c, s)
        nxt = c + _NSETS - 1
        if c >= 1 and nxt < _CHUNKS:
            for d in out_d.pop(c - 1):
                d.wait()
            in_d[nxt] = issue_in(nxt, nxt % _NSETS)
    for c in sorted(out_d):
        for d in out_d[c]:
            d.wait()


_sc_add = functools.partial(
    pl.kernel,
    mesh=plsc.VectorSubcoreMesh(core_axis_name="c", subcore_axis_name="s"),
    out_type=jax.ShapeDtypeStruct((_B * _S, _D), jnp.float32),
    scratch_types=(
        [pltpu.VMEM((_CS, _D), jnp.float32) for _ in range(_NSETS * 5)]
        + [pltpu.SemaphoreType.DMA for _ in range(_NSETS * 2)]),
    compiler_params=pltpu.CompilerParams(use_tc_tiling_on_sc=True),
)(_sc_body)


def kernel(x, pos_embedding):
    B, S, D = x.shape
    out = _sc_add(x.reshape(B * S, D), pos_embedding[:S])
    return out.reshape(B, S, D)


# speedup vs baseline: 2.9356x; 1.0015x over previous
import functools
import jax
import jax.numpy as jnp
from jax import lax
from jax.experimental import pallas as pl
from jax.experimental.pallas import tpu as pltpu
from jax.experimental.pallas import tpu_sc as plsc

_B, _S, _D = 4, 2048, 1024
_NC, _NS = 2, 16
_NW = _NC * _NS
_ROWS_W = _S // _NW
_CS = 8
_CHUNKS = _ROWS_W // _CS
_NSETS = 3
_NCOL = _D // 16


def _sc_body(x_hbm, pos_hbm, out_hbm, *scratch):
    bufs = scratch[: _NSETS * 5]
    sems = scratch[_NSETS * 5:]
    sets = []
    for s in range(_NSETS):
        pos_v = bufs[s * 5]
        xb = bufs[s * 5 + 1: s * 5 + 5]
        sets.append((pos_v, xb, sems[2 * s], sems[2 * s + 1]))

    w = lax.axis_index("s") * _NC + lax.axis_index("c")
    base_row = w * _ROWS_W

    def issue_in(c, s):
        pos_v, xb, sin, _ = sets[s]
        row0 = base_row + c * _CS
        ds = [pltpu.async_copy(pos_hbm.at[pl.ds(row0, _CS), :], pos_v, sin)]
        for b in range(_B):
            ds.append(pltpu.async_copy(
                x_hbm.at[pl.ds(b * _S + row0, _CS), :], xb[b], sin))
        return ds

    def issue_out(c, s):
        _, xb, _, sout = sets[s]
        row0 = base_row + c * _CS
        return [pltpu.async_copy(
            xb[b], out_hbm.at[pl.ds(b * _S + row0, _CS), :], sout)
            for b in range(_B)]

    def compute(s):
        pos_v, xb, _, _ = sets[s]
        for r in range(_CS):
            @plsc.parallel_loop(0, _NCOL, 1, unroll=4)
            def col_body(j):
                sl = pl.ds(j * 16, 16)
                p = pos_v[r, sl]
                for b in range(_B):
                    xb[b][r, sl] = xb[b][r, sl] + p

    in_d = {}
    out_d = {}
    for c in range(_NSETS):
        in_d[c] = issue_in(c, c % _NSETS)
    for c in range(_CHUNKS):
        s = c % _NSETS
        for d in in_d.pop(c):
            d.wait()
        compute(s)
        out_d[c] = issue_out(c, s)
        nxt = c + _NSETS - 1
        if c >= 1 and nxt < _CHUNKS:
            for d in out_d.pop(c - 1):
                d.wait()
            in_d[nxt] = issue_in(nxt, nxt % _NSETS)
    for c in sorted(out_d):
        for d in out_d[c]:
            d.wait()


_sc_add = functools.partial(
    pl.kernel,
    mesh=plsc.VectorSubcoreMesh(core_axis_name="c", subcore_axis_name="s"),
    out_type=jax.ShapeDtypeStruct((_B * _S, _D), jnp.float32),
    scratch_types=(
        [pltpu.VMEM((_CS, _D), jnp.float32) for _ in range(_NSETS * 5)]
        + [pltpu.SemaphoreType.DMA for _ in range(_NSETS * 2)]),
    compiler_params=pltpu.CompilerParams(
        use_tc_tiling_on_sc=True,
        skip_device_barrier=True,
        disable_bounds_checks=True,
        disable_semaphore_checks=True,
    ),
)(_sc_body)


def kernel(x, pos_embedding):
    B, S, D = x.shape
    out = _sc_add(x.reshape(B * S, D), pos_embedding[:S])
    return out.reshape(B, S, D)


# speedup vs baseline: 2.9772x; 1.0142x over previous
import functools
import jax
import jax.numpy as jnp
from jax import lax
from jax.experimental import pallas as pl
from jax.experimental.pallas import tpu as pltpu
from jax.experimental.pallas import tpu_sc as plsc

_B, _S, _D = 4, 2048, 1024
_NC, _NS = 2, 16
_NW = _NC * _NS
_ROWS_W = _S // _NW
_CS = 8
_CHUNKS = _ROWS_W // _CS
_NSETS = 3
_NCOL = _D // 16


def _sc_body(x_hbm, pos_hbm, out_hbm, buf, sems):
    def pos_ref(s):
        return buf.at[pl.ds(s * 5 * _CS, _CS), :]

    def x_ref(s, b):
        return buf.at[pl.ds((s * 5 + 1 + b) * _CS, _CS), :]

    w = lax.axis_index("s") * _NC + lax.axis_index("c")
    base_row = w * _ROWS_W

    def issue_in(c, s):
        row0 = base_row + c * _CS
        ds = [pltpu.async_copy(
            pos_hbm.at[pl.ds(row0, _CS), :], pos_ref(s), sems.at[2 * s])]
        for b in range(_B):
            ds.append(pltpu.async_copy(
                x_hbm.at[pl.ds(b * _S + row0, _CS), :], x_ref(s, b),
                sems.at[2 * s]))
        return ds

    def issue_out(c, s):
        row0 = base_row + c * _CS
        return [pltpu.async_copy(
            x_ref(s, b), out_hbm.at[pl.ds(b * _S + row0, _CS), :],
            sems.at[2 * s + 1])
            for b in range(_B)]

    def compute(s):
        base = s * 5 * _CS
        for r in range(_CS):
            @plsc.parallel_loop(0, _NCOL, 1, unroll=4)
            def col_body(j):
                sl = pl.ds(j * 16, 16)
                p = buf[base + r, sl]
                for b in range(_B):
                    plsc.addupdate(buf.at[base + (1 + b) * _CS + r, sl], p)

    in_d = {}
    out_d = {}
    for c in range(_NSETS):
        in_d[c] = issue_in(c, c % _NSETS)
    for c in range(_CHUNKS):
        s = c % _NSETS
        for d in in_d.pop(c):
            d.wait()
        compute(s)
        out_d[c] = issue_out(c, s)
        nxt = c + _NSETS - 1
        if c >= 1 and nxt < _CHUNKS:
            for d in out_d.pop(c - 1):
                d.wait()
            in_d[nxt] = issue_in(nxt, nxt % _NSETS)
    for c in sorted(out_d):
        for d in out_d[c]:
            d.wait()


_sc_add = functools.partial(
    pl.kernel,
    mesh=plsc.VectorSubcoreMesh(core_axis_name="c", subcore_axis_name="s"),
    out_type=jax.ShapeDtypeStruct((_B * _S, _D), jnp.float32),
    scratch_types=[
        pltpu.VMEM((_NSETS * 5 * _CS, _D), jnp.float32),
        pltpu.SemaphoreType.DMA((_NSETS * 2,)),
    ],
    compiler_params=pltpu.CompilerParams(
        use_tc_tiling_on_sc=True,
        skip_device_barrier=True,
        disable_bounds_checks=True,
        disable_semaphore_checks=True,
    ),
)(_sc_body)


def kernel(x, pos_embedding):
    B, S, D = x.shape
    out = _sc_add(x.reshape(B * S, D), pos_embedding[:S])
    return out.reshape(B, S, D)


# speedup vs baseline: 3.0200x; 1.0144x over previous
import functools
import jax
import jax.numpy as jnp
from jax import lax
from jax.experimental import pallas as pl
from jax.experimental.pallas import tpu as pltpu
from jax.experimental.pallas import tpu_sc as plsc

_B, _S, _D = 4, 2048, 1024
_NC, _NS = 2, 16
_NW = _NC * _NS
_ROWS_W = _S // _NW
_CS = 8
_CHUNKS = _ROWS_W // _CS
_NSETS = 3
_NCOL = _D // 16


def _sc_body(x_hbm, pos_hbm, out_hbm, pos_buf, x_buf, sems):
    w = lax.axis_index("s") * _NC + lax.axis_index("c")
    base_row = w * _ROWS_W

    def issue_in(c, s):
        row0 = base_row + c * _CS
        return [
            pltpu.async_copy(
                pos_hbm.at[pl.ds(row0, _CS), :],
                pos_buf.at[pl.ds(s * _CS, _CS), :], sems.at[2 * s]),
            pltpu.async_copy(
                x_hbm.at[:, pl.ds(row0, _CS), :], x_buf.at[s], sems.at[2 * s]),
        ]

    def issue_out(c, s):
        row0 = base_row + c * _CS
        return [pltpu.async_copy(
            x_buf.at[s], out_hbm.at[:, pl.ds(row0, _CS), :],
            sems.at[2 * s + 1])]

    def compute(s):
        for r in range(_CS):
            @plsc.parallel_loop(0, _NCOL, 1, unroll=4)
            def col_body(j):
                sl = pl.ds(j * 16, 16)
                p = pos_buf[s * _CS + r, sl]
                for b in range(_B):
                    plsc.addupdate(x_buf.at[s, b, r, sl], p)

    in_d = {}
    out_d = {}
    for c in range(_NSETS):
        in_d[c] = issue_in(c, c % _NSETS)
    for c in range(_CHUNKS):
        s = c % _NSETS
        for d in in_d.pop(c):
            d.wait()
        compute(s)
        out_d[c] = issue_out(c, s)
        nxt = c + _NSETS - 1
        if c >= 1 and nxt < _CHUNKS:
            for d in out_d.pop(c - 1):
                d.wait()
            in_d[nxt] = issue_in(nxt, nxt % _NSETS)
    for c in sorted(out_d):
        for d in out_d[c]:
            d.wait()


_sc_add = functools.partial(
    pl.kernel,
    mesh=plsc.VectorSubcoreMesh(core_axis_name="c", subcore_axis_name="s"),
    out_type=jax.ShapeDtypeStruct((_B, _S, _D), jnp.float32),
    scratch_types=[
        pltpu.VMEM((_NSETS * _CS, _D), jnp.float32),
        pltpu.VMEM((_NSETS, _B, _CS, _D), jnp.float32),
        pltpu.SemaphoreType.DMA((_NSETS * 2,)),
    ],
    compiler_params=pltpu.CompilerParams(
        use_tc_tiling_on_sc=True,
        skip_device_barrier=True,
        disable_bounds_checks=True,
        disable_semaphore_checks=True,
    ),
)(_sc_body)


def kernel(x, pos_embedding):
    B, S, D = x.shape
    return _sc_add(x, pos_embedding[:S])
